# baseline (speedup 1.0000x reference)
import functools

import jax
import jax.numpy as jnp
from jax import lax
from jax.experimental import pallas as pl
from jax.experimental.pallas import tpu as pltpu
from jax.experimental.pallas import tpu_sc as plsc


def kernel(h, r, t, ent_embeddings, rel_embeddings):
    B = h.shape[0]
    D = ent_embeddings.shape[1]
    H = D // 2
    info = plsc.get_sparse_core_info()
    NC, NS = info.num_cores, info.num_subcores
    NW = NC * NS
    b_per_w = B // NW

    ent_lo = ent_embeddings[:, :H]
    ent_hi = ent_embeddings[:, H:]

    mesh = plsc.VectorSubcoreMesh(core_axis_name="c", subcore_axis_name="s")
    out_t = jax.ShapeDtypeStruct((B, D), jnp.float32)

    @functools.partial(
        pl.kernel,
        mesh=mesh,
        out_type=[out_t, out_t, out_t],
        compiler_params=pltpu.CompilerParams(use_tc_tiling_on_sc=False),
        scratch_types=[
            pltpu.VMEM((b_per_w,), jnp.int32),
            pltpu.VMEM((b_per_w,), jnp.int32),
            pltpu.VMEM((b_per_w,), jnp.int32),
            pltpu.VMEM((b_per_w, H), jnp.float32),
            pltpu.VMEM((b_per_w, H), jnp.float32),
            pltpu.VMEM((b_per_w, D), jnp.float32),
            pltpu.SemaphoreType.DMA,
            pltpu.SemaphoreType.DMA,
            pltpu.SemaphoreType.DMA,
        ],
    )
    def gather3(h_hbm, r_hbm, t_hbm, lo_hbm, hi_hbm, rel_hbm, oh, ot, orr,
                h_v, r_v, t_v, lo_rows, hi_rows, r_rows,
                sem_lo, sem_hi, sem_r):
        wid = lax.axis_index("s") * NC + lax.axis_index("c")
        base = wid * b_per_w
        pltpu.sync_copy(h_hbm.at[pl.ds(base, b_per_w)], h_v)
        pltpu.sync_copy(t_hbm.at[pl.ds(base, b_per_w)], t_v)
        pltpu.sync_copy(r_hbm.at[pl.ds(base, b_per_w)], r_v)

        cr = pltpu.async_copy(rel_hbm.at[r_v], r_rows, sem_r)

        for idx_v, out_hbm in ((h_v, oh), (t_v, ot)):
            clo = pltpu.async_copy(lo_hbm.at[idx_v], lo_rows, sem_lo)
            chi = pltpu.async_copy(hi_hbm.at[idx_v], hi_rows, sem_hi)
            clo.wait()
            pltpu.sync_copy(
                lo_rows, out_hbm.at[pl.ds(base, b_per_w), pl.ds(0, H)])
            chi.wait()
            pltpu.sync_copy(
                hi_rows, out_hbm.at[pl.ds(base, b_per_w), pl.ds(H, H)])

        cr.wait()
        pltpu.sync_copy(r_rows, orr.at[pl.ds(base, b_per_w)])

    h_e, t_e, r_e = gather3(h, r, t, ent_lo, ent_hi, rel_embeddings)
    return (h_e, t_e, r_e)
